# Initial kernel scaffold; baseline (speedup 1.0000x reference)
import jax
import jax.numpy as jnp
from jax.experimental import pallas as pl


def kernel(x, edge_index, adj_values, W, b):
    raise NotImplementedError("write your pallas kernel here")


# speedup vs baseline: 1.9830x; 1.9830x over previous
import functools

import jax
import jax.numpy as jnp
from jax import lax
from jax.experimental import pallas as pl
from jax.experimental.pallas import tpu as pltpu
from jax.experimental.pallas import tpu_sc as plsc

MIN_NORM = 1e-15
EPS = 4e-3
N = 10000
E = 320000
D = 128

NC = 2
NS = 16
NPAD = 10240
ROWS_PT = NPAD // NS
EPW = E // (NC * NS)
BE = 32
EPWP = 10016
NBLK = EPWP // BE

BM = 1000


def _artanh(x):
    x = jnp.clip(x, -1.0 + 1e-7, 1.0 - 1e-7)
    return 0.5 * (jnp.log1p(x) - jnp.log1p(-x))


def _norm(x):
    return jnp.clip(jnp.sqrt(jnp.sum(x * x, axis=-1, keepdims=True)), MIN_NORM, None)


def _proj(x):
    norm = _norm(x)
    maxnorm = 1.0 - EPS
    return jnp.where(norm > maxnorm, x / norm * maxnorm, x)


def _expmap0(u):
    un = _norm(u)
    return jnp.tanh(un) * u / un


def _logmap0(p):
    pn = _norm(p)
    return _artanh(pn) / pn * p


def _mobius_add(x, y):
    x2 = jnp.sum(x * x, axis=-1, keepdims=True)
    y2 = jnp.sum(y * y, axis=-1, keepdims=True)
    xy = jnp.sum(x * y, axis=-1, keepdims=True)
    num = (1.0 + 2.0 * xy + y2) * x + (1.0 - x2) * y
    den = 1.0 + 2.0 * xy + x2 * y2
    return num / jnp.clip(den, MIN_NORM, None)


def _pre_body(x_ref, wt_ref, b_ref, o_ref):
    x = x_ref[...]
    mx = jnp.dot(x, wt_ref[...], preferred_element_type=jnp.float32)
    xn = _norm(x)
    mxn = _norm(mx)
    res_c = jnp.tanh(mxn / xn * _artanh(xn)) * mx / mxn
    cond = jnp.all(mx == 0.0, axis=-1, keepdims=True)
    res = _proj(jnp.where(cond, jnp.zeros_like(res_c), res_c))
    hyp_bias = _proj(_expmap0(b_ref[...]))
    res = _proj(_mobius_add(res, hyp_bias))
    o_ref[...] = _logmap0(res)


_pre = pl.pallas_call(
    _pre_body,
    grid=(N // BM,),
    in_specs=[
        pl.BlockSpec((BM, D), lambda i: (i, 0)),
        pl.BlockSpec((D, D), lambda i: (0, 0)),
        pl.BlockSpec((1, D), lambda i: (0, 0)),
    ],
    out_specs=pl.BlockSpec((BM, D), lambda i: (i, 0)),
    out_shape=jax.ShapeDtypeStruct((N, D), jnp.float32),
)


def _post_body(p0_ref, p1_ref, o_ref):
    s = p0_ref[...][0] + p1_ref[...][0]
    h = _proj(_expmap0(s))
    xt = jnp.maximum(_logmap0(h), 0.0)
    o_ref[...] = _proj(_expmap0(xt))


_post = pl.pallas_call(
    _post_body,
    grid=(N // BM,),
    in_specs=[
        pl.BlockSpec((1, BM, D), lambda i: (0, i, 0)),
        pl.BlockSpec((1, BM, D), lambda i: (1, i, 0)),
    ],
    out_specs=pl.BlockSpec((BM, D), lambda i: (i, 0)),
    out_shape=jax.ShapeDtypeStruct((N, D), jnp.float32),
)


def _spmm_body(xt_hbm, src_hbm, dst_hbm, adj_hbm, out_hbm,
               src_v, dst_v, adj_v, rows_v, work_v, acc_sh, sem_g):
    c = lax.axis_index("c")
    s = lax.axis_index("s")

    zeros16 = jnp.zeros((16,), jnp.float32)
    for r in range(BE):
        for h in range(D // 16):
            work_v[r, pl.ds(h * 16, 16)] = zeros16
    for z in range(ROWS_PT // BE):
        pltpu.sync_copy(work_v, acc_sh.at[pl.ds(s * ROWS_PT + z * BE, BE)])
    plsc.subcore_barrier()

    def block(i, carry):
        pltpu.sync_copy(src_hbm.at[c, s, pl.ds(i * BE, BE)], src_v)
        pltpu.sync_copy(dst_hbm.at[c, s, pl.ds(i * BE, BE)], dst_v)
        pltpu.sync_copy(adj_hbm.at[c, s, pl.ds(i * BE, BE)], adj_v)
        pltpu.async_copy(xt_hbm.at[src_v], rows_v, sem_g).wait()
        for r in range(BE):
            a = adj_v[r, pl.ds(0, 16)]
            for h in range(D // 16):
                work_v[r, pl.ds(h * 16, 16)] = rows_v[r, pl.ds(h * 16, 16)] * a
        pltpu.sync_copy(work_v, acc_sh.at[dst_v], add=True)
        return carry

    lax.fori_loop(0, NBLK, block, 0)
    plsc.subcore_barrier()
    pltpu.sync_copy(acc_sh.at[pl.ds(s * ROWS_PT, ROWS_PT)],
                    out_hbm.at[c, pl.ds(s * ROWS_PT, ROWS_PT)])


@functools.lru_cache(maxsize=None)
def _get_spmm():
    return functools.partial(
        pl.kernel,
        out_type=jax.ShapeDtypeStruct((NC, NPAD, D), jnp.float32),
        mesh=plsc.VectorSubcoreMesh(core_axis_name="c", subcore_axis_name="s"),
        scratch_types=[
            pltpu.VMEM((BE,), jnp.int32),
            pltpu.VMEM((BE,), jnp.int32),
            pltpu.VMEM((BE, 16), jnp.float32),
            pltpu.VMEM((BE, D), jnp.float32),
            pltpu.VMEM((BE, D), jnp.float32),
            pltpu.VMEM_SHARED((NPAD, D), jnp.float32),
            pltpu.SemaphoreType.DMA,
        ],
    )(_spmm_body)


def kernel(x, edge_index, adj_values, W, b):
    xt = _pre(x, W.T, b.reshape(1, D))
    npad = NC * NS * EPWP - E
    src = jnp.pad(edge_index[0], (0, npad)).reshape(NC, NS, EPWP)
    dst = jnp.pad(edge_index[1], (0, npad)).reshape(NC, NS, EPWP)
    adj = jnp.pad(adj_values, (0, npad)).reshape(NC, NS, EPWP, 1)
    adj = jnp.broadcast_to(adj, (NC, NS, EPWP, 16))
    partials = _get_spmm()(xt, src, dst, adj)
    return _post(partials, partials)


# speedup vs baseline: 2.4976x; 1.2595x over previous
import functools

import jax
import jax.numpy as jnp
from jax import lax
from jax.experimental import pallas as pl
from jax.experimental.pallas import tpu as pltpu
from jax.experimental.pallas import tpu_sc as plsc

MIN_NORM = 1e-15
EPS = 4e-3
N = 10000
E = 320000
D = 128

NC = 2
NS = 16
NPAD = 10240
ROWS_PT = NPAD // NS
EPW = E // (NC * NS)
BE = 32
EPWP = 10048
NBLK = EPWP // BE
PAIRS = NBLK // 2

BM = 1000


def _artanh(x):
    x = jnp.clip(x, -1.0 + 1e-7, 1.0 - 1e-7)
    return 0.5 * (jnp.log1p(x) - jnp.log1p(-x))


def _norm(x):
    return jnp.clip(jnp.sqrt(jnp.sum(x * x, axis=-1, keepdims=True)), MIN_NORM, None)


def _proj(x):
    norm = _norm(x)
    maxnorm = 1.0 - EPS
    return jnp.where(norm > maxnorm, x / norm * maxnorm, x)


def _expmap0(u):
    un = _norm(u)
    return jnp.tanh(un) * u / un


def _logmap0(p):
    pn = _norm(p)
    return _artanh(pn) / pn * p


def _mobius_add(x, y):
    x2 = jnp.sum(x * x, axis=-1, keepdims=True)
    y2 = jnp.sum(y * y, axis=-1, keepdims=True)
    xy = jnp.sum(x * y, axis=-1, keepdims=True)
    num = (1.0 + 2.0 * xy + y2) * x + (1.0 - x2) * y
    den = 1.0 + 2.0 * xy + x2 * y2
    return num / jnp.clip(den, MIN_NORM, None)


def _pre_body(x_ref, wt_ref, b_ref, o_ref):
    x = x_ref[...]
    mx = jnp.dot(x, wt_ref[...], preferred_element_type=jnp.float32)
    xn = _norm(x)
    mxn = _norm(mx)
    res_c = jnp.tanh(mxn / xn * _artanh(xn)) * mx / mxn
    cond = jnp.all(mx == 0.0, axis=-1, keepdims=True)
    res = _proj(jnp.where(cond, jnp.zeros_like(res_c), res_c))
    hyp_bias = _proj(_expmap0(b_ref[...]))
    res = _proj(_mobius_add(res, hyp_bias))
    o_ref[...] = _logmap0(res)


_pre = pl.pallas_call(
    _pre_body,
    grid=(N // BM,),
    in_specs=[
        pl.BlockSpec((BM, D), lambda i: (i, 0)),
        pl.BlockSpec((D, D), lambda i: (0, 0)),
        pl.BlockSpec((1, D), lambda i: (0, 0)),
    ],
    out_specs=pl.BlockSpec((BM, D), lambda i: (i, 0)),
    out_shape=jax.ShapeDtypeStruct((N, D), jnp.float32),
)


def _post_body(p0_ref, p1_ref, o_ref):
    s = p0_ref[...][0] + p1_ref[...][0]
    h = _proj(_expmap0(s))
    xt = jnp.maximum(_logmap0(h), 0.0)
    o_ref[...] = _proj(_expmap0(xt))


_post = pl.pallas_call(
    _post_body,
    grid=(N // BM,),
    in_specs=[
        pl.BlockSpec((1, BM, D), lambda i: (0, i, 0)),
        pl.BlockSpec((1, BM, D), lambda i: (1, i, 0)),
    ],
    out_specs=pl.BlockSpec((BM, D), lambda i: (i, 0)),
    out_shape=jax.ShapeDtypeStruct((N, D), jnp.float32),
)


def _spmm_body(xt_hbm, src_hbm, dst_hbm, adj_hbm, out_hbm,
               src0, dst0, adj0, rows0, src1, dst1, adj1, rows1,
               work_v, acc_sh, sg0, sg1):
    c = lax.axis_index("c")
    s = lax.axis_index("s")

    zeros16 = jnp.zeros((16,), jnp.float32)
    for r in range(BE):
        for h in range(D // 16):
            work_v[r, pl.ds(h * 16, 16)] = zeros16
    for z in range(ROWS_PT // BE):
        pltpu.sync_copy(work_v, acc_sh.at[pl.ds(s * ROWS_PT + z * BE, BE)])
    rem = ROWS_PT - (ROWS_PT // BE) * BE
    if rem:
        pltpu.sync_copy(
            work_v.at[pl.ds(0, rem)],
            acc_sh.at[pl.ds(s * ROWS_PT + (ROWS_PT // BE) * BE, rem)])
    plsc.subcore_barrier()

    def scale_and_scatter(rows_v, adj_v, dst_v):
        for r in range(BE):
            a = adj_v[r, pl.ds(0, 16)]
            for h in range(D // 16):
                work_v[r, pl.ds(h * 16, 16)] = rows_v[r, pl.ds(h * 16, 16)] * a
        pltpu.sync_copy(work_v, acc_sh.at[dst_v], add=True)

    def pair(p, carry):
        i0 = 2 * p
        i1 = 2 * p + 1
        pltpu.sync_copy(src_hbm.at[c, s, pl.ds(i0 * BE, BE)], src0)
        pltpu.sync_copy(dst_hbm.at[c, s, pl.ds(i0 * BE, BE)], dst0)
        pltpu.sync_copy(adj_hbm.at[c, s, pl.ds(i0 * BE, BE)], adj0)
        g0 = pltpu.async_copy(xt_hbm.at[src0], rows0, sg0)
        pltpu.sync_copy(src_hbm.at[c, s, pl.ds(i1 * BE, BE)], src1)
        pltpu.sync_copy(dst_hbm.at[c, s, pl.ds(i1 * BE, BE)], dst1)
        pltpu.sync_copy(adj_hbm.at[c, s, pl.ds(i1 * BE, BE)], adj1)
        g1 = pltpu.async_copy(xt_hbm.at[src1], rows1, sg1)
        g0.wait()
        scale_and_scatter(rows0, adj0, dst0)
        g1.wait()
        scale_and_scatter(rows1, adj1, dst1)
        return carry

    lax.fori_loop(0, PAIRS, pair, 0)
    plsc.subcore_barrier()
    pltpu.sync_copy(acc_sh.at[pl.ds(s * ROWS_PT, ROWS_PT)],
                    out_hbm.at[c, pl.ds(s * ROWS_PT, ROWS_PT)])


@functools.lru_cache(maxsize=None)
def _get_spmm():
    return functools.partial(
        pl.kernel,
        out_type=jax.ShapeDtypeStruct((NC, NPAD, D), jnp.float32),
        mesh=plsc.VectorSubcoreMesh(core_axis_name="c", subcore_axis_name="s"),
        scratch_types=[
            pltpu.VMEM((BE,), jnp.int32),
            pltpu.VMEM((BE,), jnp.int32),
            pltpu.VMEM((BE, 16), jnp.float32),
            pltpu.VMEM((BE, D), jnp.float32),
            pltpu.VMEM((BE,), jnp.int32),
            pltpu.VMEM((BE,), jnp.int32),
            pltpu.VMEM((BE, 16), jnp.float32),
            pltpu.VMEM((BE, D), jnp.float32),
            pltpu.VMEM((BE, D), jnp.float32),
            pltpu.VMEM_SHARED((NPAD, D), jnp.float32),
            pltpu.SemaphoreType.DMA,
            pltpu.SemaphoreType.DMA,
        ],
    )(_spmm_body)


def kernel(x, edge_index, adj_values, W, b):
    xt = _pre(x, W.T, b.reshape(1, D))
    npad = NC * NS * EPWP - E
    src = jnp.pad(edge_index[0], (0, npad)).reshape(NC, NS, EPWP)
    dst = jnp.pad(edge_index[1], (0, npad)).reshape(NC, NS, EPWP)
    adj = jnp.pad(adj_values, (0, npad)).reshape(NC, NS, EPWP, 1)
    adj = jnp.broadcast_to(adj, (NC, NS, EPWP, 16))
    partials = _get_spmm()(xt, src, dst, adj)
    return _post(partials, partials)


# speedup vs baseline: 2.8752x; 1.1512x over previous
import functools

import jax
import jax.numpy as jnp
from jax import lax
from jax.experimental import pallas as pl
from jax.experimental.pallas import tpu as pltpu
from jax.experimental.pallas import tpu_sc as plsc

MIN_NORM = 1e-15
EPS = 4e-3
N = 10000
E = 320000
D = 128

NC = 2
NS = 16
NPAD = 10240
ROWS_PT = NPAD // NS
EPW = E // (NC * NS)
BE = 32
EPWP = 10048
NBLK = EPWP // BE
PAIRS = NBLK // 2

BM = 1000


def _artanh(x):
    x = jnp.clip(x, -1.0 + 1e-7, 1.0 - 1e-7)
    return 0.5 * (jnp.log1p(x) - jnp.log1p(-x))


def _norm(x):
    return jnp.clip(jnp.sqrt(jnp.sum(x * x, axis=-1, keepdims=True)), MIN_NORM, None)


def _proj(x):
    norm = _norm(x)
    maxnorm = 1.0 - EPS
    return jnp.where(norm > maxnorm, x / norm * maxnorm, x)


def _expmap0(u):
    un = _norm(u)
    return jnp.tanh(un) * u / un


def _logmap0(p):
    pn = _norm(p)
    return _artanh(pn) / pn * p


def _mobius_add(x, y):
    x2 = jnp.sum(x * x, axis=-1, keepdims=True)
    y2 = jnp.sum(y * y, axis=-1, keepdims=True)
    xy = jnp.sum(x * y, axis=-1, keepdims=True)
    num = (1.0 + 2.0 * xy + y2) * x + (1.0 - x2) * y
    den = 1.0 + 2.0 * xy + x2 * y2
    return num / jnp.clip(den, MIN_NORM, None)


def _pre_body(x_ref, wt_ref, b_ref, o_ref):
    x = x_ref[...]
    mx = jnp.dot(x, wt_ref[...], preferred_element_type=jnp.float32)
    xn = _norm(x)
    mxn = _norm(mx)
    res_c = jnp.tanh(mxn / xn * _artanh(xn)) * mx / mxn
    cond = jnp.all(mx == 0.0, axis=-1, keepdims=True)
    res = _proj(jnp.where(cond, jnp.zeros_like(res_c), res_c))
    hyp_bias = _proj(_expmap0(b_ref[...]))
    res = _proj(_mobius_add(res, hyp_bias))
    o_ref[...] = _logmap0(res)


_pre = pl.pallas_call(
    _pre_body,
    grid=(N // BM,),
    in_specs=[
        pl.BlockSpec((BM, D), lambda i: (i, 0)),
        pl.BlockSpec((D, D), lambda i: (0, 0)),
        pl.BlockSpec((1, D), lambda i: (0, 0)),
    ],
    out_specs=pl.BlockSpec((BM, D), lambda i: (i, 0)),
    out_shape=jax.ShapeDtypeStruct((N, D), jnp.float32),
)


def _post_body(p0_ref, p1_ref, o_ref):
    s = p0_ref[...][0] + p1_ref[...][0]
    h = _proj(_expmap0(s))
    xt = jnp.maximum(_logmap0(h), 0.0)
    o_ref[...] = _proj(_expmap0(xt))


_post = pl.pallas_call(
    _post_body,
    grid=(N // BM,),
    in_specs=[
        pl.BlockSpec((1, BM, D), lambda i: (0, i, 0)),
        pl.BlockSpec((1, BM, D), lambda i: (1, i, 0)),
    ],
    out_specs=pl.BlockSpec((BM, D), lambda i: (i, 0)),
    out_shape=jax.ShapeDtypeStruct((N, D), jnp.float32),
)


def _spmm_body(xt_hbm, src_hbm, dst_hbm, adj_hbm, out_hbm,
               src01, dst01, adj01, rows0, rows1,
               work_v, acc_sh, sg0, sg1):
    c = lax.axis_index("c")
    s = lax.axis_index("s")

    zeros16 = jnp.zeros((16,), jnp.float32)
    for r in range(BE):
        for h in range(D // 16):
            work_v[r, pl.ds(h * 16, 16)] = zeros16
    for z in range(ROWS_PT // BE):
        pltpu.sync_copy(work_v, acc_sh.at[pl.ds(s * ROWS_PT + z * BE, BE)])
    rem = ROWS_PT - (ROWS_PT // BE) * BE
    if rem:
        pltpu.sync_copy(
            work_v.at[pl.ds(0, rem)],
            acc_sh.at[pl.ds(s * ROWS_PT + (ROWS_PT // BE) * BE, rem)])
    plsc.subcore_barrier()

    def scale_and_scatter(rows_v, aoff, dst_ref):
        for r in range(BE):
            a = adj01[aoff + r, pl.ds(0, 16)]
            for h in range(D // 16):
                work_v[r, pl.ds(h * 16, 16)] = rows_v[r, pl.ds(h * 16, 16)] * a
        pltpu.sync_copy(work_v, acc_sh.at[dst_ref], add=True)

    def pair(p, carry):
        pltpu.sync_copy(src_hbm.at[c, s, p], src01)
        pltpu.sync_copy(dst_hbm.at[c, s, p], dst01)
        pltpu.sync_copy(adj_hbm.at[c, s, p], adj01)
        g0 = pltpu.async_copy(xt_hbm.at[src01.at[0]], rows0, sg0)
        g1 = pltpu.async_copy(xt_hbm.at[src01.at[1]], rows1, sg1)
        g0.wait()
        scale_and_scatter(rows0, 0, dst01.at[0])
        g1.wait()
        scale_and_scatter(rows1, BE, dst01.at[1])
        return carry

    lax.fori_loop(0, PAIRS, pair, 0)
    plsc.subcore_barrier()
    pltpu.sync_copy(acc_sh.at[pl.ds(s * ROWS_PT, ROWS_PT)],
                    out_hbm.at[c, pl.ds(s * ROWS_PT, ROWS_PT)])


@functools.lru_cache(maxsize=None)
def _get_spmm():
    return functools.partial(
        pl.kernel,
        out_type=jax.ShapeDtypeStruct((NC, NPAD, D), jnp.float32),
        mesh=plsc.VectorSubcoreMesh(core_axis_name="c", subcore_axis_name="s"),
        scratch_types=[
            pltpu.VMEM((2, BE), jnp.int32),
            pltpu.VMEM((2, BE), jnp.int32),
            pltpu.VMEM((2 * BE, 16), jnp.float32),
            pltpu.VMEM((BE, D), jnp.float32),
            pltpu.VMEM((BE, D), jnp.float32),
            pltpu.VMEM((BE, D), jnp.float32),
            pltpu.VMEM_SHARED((NPAD, D), jnp.float32),
            pltpu.SemaphoreType.DMA,
            pltpu.SemaphoreType.DMA,
        ],
    )(_spmm_body)


def kernel(x, edge_index, adj_values, W, b):
    xt = _pre(x, W.T, b.reshape(1, D))
    npad = NC * NS * EPWP - E
    src = jnp.pad(edge_index[0], (0, npad)).reshape(NC, NS, PAIRS, 2, BE)
    dst = jnp.pad(edge_index[1], (0, npad)).reshape(NC, NS, PAIRS, 2, BE)
    adj = jnp.pad(adj_values, (0, npad)).reshape(NC, NS, PAIRS, 2 * BE, 1)
    adj = jnp.broadcast_to(adj, (NC, NS, PAIRS, 2 * BE, 16))
    partials = _get_spmm()(xt, src, dst, adj)
    return _post(partials, partials)


# speedup vs baseline: 3.4427x; 1.1974x over previous
import functools

import jax
import jax.numpy as jnp
from jax import lax
from jax.experimental import pallas as pl
from jax.experimental.pallas import tpu as pltpu
from jax.experimental.pallas import tpu_sc as plsc

MIN_NORM = 1e-15
EPS = 4e-3
N = 10000
E = 320000
D = 128

NC = 2
NS = 16
NPAD = 10240
ROWS_PT = NPAD // NS
EPW = E // (NC * NS)
BE = 32
EPWP = 10048
NBLK = EPWP // BE
PAIRS = NBLK // 2

BM = 1000


def _artanh(x):
    x = jnp.clip(x, -1.0 + 1e-7, 1.0 - 1e-7)
    return 0.5 * (jnp.log1p(x) - jnp.log1p(-x))


def _norm(x):
    return jnp.clip(jnp.sqrt(jnp.sum(x * x, axis=-1, keepdims=True)), MIN_NORM, None)


def _proj(x):
    norm = _norm(x)
    maxnorm = 1.0 - EPS
    return jnp.where(norm > maxnorm, x / norm * maxnorm, x)


def _expmap0(u):
    un = _norm(u)
    return jnp.tanh(un) * u / un


def _logmap0(p):
    pn = _norm(p)
    return _artanh(pn) / pn * p


def _mobius_add(x, y):
    x2 = jnp.sum(x * x, axis=-1, keepdims=True)
    y2 = jnp.sum(y * y, axis=-1, keepdims=True)
    xy = jnp.sum(x * y, axis=-1, keepdims=True)
    num = (1.0 + 2.0 * xy + y2) * x + (1.0 - x2) * y
    den = 1.0 + 2.0 * xy + x2 * y2
    return num / jnp.clip(den, MIN_NORM, None)


def _pre_body(x_ref, wt_ref, b_ref, o_ref):
    x = x_ref[...]
    mx = jnp.dot(x, wt_ref[...], preferred_element_type=jnp.float32)
    xn = _norm(x)
    mxn = _norm(mx)
    res_c = jnp.tanh(mxn / xn * _artanh(xn)) * mx / mxn
    cond = jnp.all(mx == 0.0, axis=-1, keepdims=True)
    res = _proj(jnp.where(cond, jnp.zeros_like(res_c), res_c))
    hyp_bias = _proj(_expmap0(b_ref[...]))
    res = _proj(_mobius_add(res, hyp_bias))
    o_ref[...] = _logmap0(res)


_pre = pl.pallas_call(
    _pre_body,
    grid=(N // BM,),
    in_specs=[
        pl.BlockSpec((BM, D), lambda i: (i, 0)),
        pl.BlockSpec((D, D), lambda i: (0, 0)),
        pl.BlockSpec((1, D), lambda i: (0, 0)),
    ],
    out_specs=pl.BlockSpec((BM, D), lambda i: (i, 0)),
    out_shape=jax.ShapeDtypeStruct((N, D), jnp.float32),
)


def _post_body(p0_ref, p1_ref, o_ref):
    s = p0_ref[...][0] + p1_ref[...][0]
    h = _proj(_expmap0(s))
    xt = jnp.maximum(_logmap0(h), 0.0)
    o_ref[...] = _proj(_expmap0(xt))


_post = pl.pallas_call(
    _post_body,
    grid=(N // BM,),
    in_specs=[
        pl.BlockSpec((1, BM, D), lambda i: (0, i, 0)),
        pl.BlockSpec((1, BM, D), lambda i: (1, i, 0)),
    ],
    out_specs=pl.BlockSpec((BM, D), lambda i: (i, 0)),
    out_shape=jax.ShapeDtypeStruct((N, D), jnp.float32),
)


def _spmm_body(xt_hbm, src_hbm, dst_hbm, adj_hbm, out_hbm,
               src01, dst01, adj01, rows0, rows1,
               work_v, acc_sh, sg0, sg1):
    c = lax.axis_index("c")
    s = lax.axis_index("s")

    zeros16 = jnp.zeros((16,), jnp.float32)
    for r in range(BE):
        for h in range(D // 16):
            work_v[r, pl.ds(h * 16, 16)] = zeros16
    for z in range(ROWS_PT // BE):
        pltpu.sync_copy(work_v, acc_sh.at[pl.ds(s * ROWS_PT + z * BE, BE)])
    rem = ROWS_PT - (ROWS_PT // BE) * BE
    if rem:
        pltpu.sync_copy(
            work_v.at[pl.ds(0, rem)],
            acc_sh.at[pl.ds(s * ROWS_PT + (ROWS_PT // BE) * BE, rem)])
    plsc.subcore_barrier()

    def scale_and_scatter(rows_v, blk, dst_ref):
        for g in range(BE // 16):
            av = adj01[blk, pl.ds(g * 16, 16)]
            for u in range(16):
                r = g * 16 + u
                a = jnp.full((16,), av[u], jnp.float32)
                for h in range(D // 16):
                    work_v[r, pl.ds(h * 16, 16)] = (
                        rows_v[r, pl.ds(h * 16, 16)] * a)
        pltpu.sync_copy(work_v, acc_sh.at[dst_ref], add=True)

    def pair(p, carry):
        pltpu.sync_copy(src_hbm.at[c, s, p], src01)
        pltpu.sync_copy(dst_hbm.at[c, s, p], dst01)
        pltpu.sync_copy(adj_hbm.at[c, s, p], adj01)
        g0 = pltpu.async_copy(xt_hbm.at[src01.at[0]], rows0, sg0)
        g1 = pltpu.async_copy(xt_hbm.at[src01.at[1]], rows1, sg1)
        g0.wait()
        scale_and_scatter(rows0, 0, dst01.at[0])
        g1.wait()
        scale_and_scatter(rows1, 1, dst01.at[1])
        return carry

    lax.fori_loop(0, PAIRS, pair, 0)
    plsc.subcore_barrier()
    pltpu.sync_copy(acc_sh.at[pl.ds(s * ROWS_PT, ROWS_PT)],
                    out_hbm.at[c, pl.ds(s * ROWS_PT, ROWS_PT)])


@functools.lru_cache(maxsize=None)
def _get_spmm():
    return functools.partial(
        pl.kernel,
        out_type=jax.ShapeDtypeStruct((NC, NPAD, D), jnp.float32),
        mesh=plsc.VectorSubcoreMesh(core_axis_name="c", subcore_axis_name="s"),
        scratch_types=[
            pltpu.VMEM((2, BE), jnp.int32),
            pltpu.VMEM((2, BE), jnp.int32),
            pltpu.VMEM((2, BE), jnp.float32),
            pltpu.VMEM((BE, D), jnp.float32),
            pltpu.VMEM((BE, D), jnp.float32),
            pltpu.VMEM((BE, D), jnp.float32),
            pltpu.VMEM_SHARED((NPAD, D), jnp.float32),
            pltpu.SemaphoreType.DMA,
            pltpu.SemaphoreType.DMA,
        ],
    )(_spmm_body)


def kernel(x, edge_index, adj_values, W, b):
    xt = _pre(x, W.T, b.reshape(1, D))
    npad = NC * NS * EPWP - E
    src = jnp.pad(edge_index[0], (0, npad)).reshape(NC, NS, PAIRS, 2, BE)
    dst = jnp.pad(edge_index[1], (0, npad)).reshape(NC, NS, PAIRS, 2, BE)
    adj = jnp.pad(adj_values, (0, npad)).reshape(NC, NS, PAIRS, 2, BE)
    partials = _get_spmm()(xt, src, dst, adj)
    return _post(partials, partials)


# speedup vs baseline: 3.6430x; 1.0582x over previous
import functools

import jax
import jax.numpy as jnp
from jax import lax
from jax.experimental import pallas as pl
from jax.experimental.pallas import tpu as pltpu
from jax.experimental.pallas import tpu_sc as plsc

MIN_NORM = 1e-15
EPS = 4e-3
N = 10000
E = 320000
D = 128

NC = 2
NS = 16
NPAD = 10240
ROWS_PT = NPAD // NS
EPW = E // (NC * NS)
BE = 32
EPWP = 10112
NBLK = EPWP // BE
QUADS = NBLK // 4

BM = 1000


def _artanh(x):
    x = jnp.clip(x, -1.0 + 1e-7, 1.0 - 1e-7)
    return 0.5 * (jnp.log1p(x) - jnp.log1p(-x))


def _norm(x):
    return jnp.clip(jnp.sqrt(jnp.sum(x * x, axis=-1, keepdims=True)), MIN_NORM, None)


def _proj(x):
    norm = _norm(x)
    maxnorm = 1.0 - EPS
    return jnp.where(norm > maxnorm, x / norm * maxnorm, x)


def _expmap0(u):
    un = _norm(u)
    return jnp.tanh(un) * u / un


def _logmap0(p):
    pn = _norm(p)
    return _artanh(pn) / pn * p


def _mobius_add(x, y):
    x2 = jnp.sum(x * x, axis=-1, keepdims=True)
    y2 = jnp.sum(y * y, axis=-1, keepdims=True)
    xy = jnp.sum(x * y, axis=-1, keepdims=True)
    num = (1.0 + 2.0 * xy + y2) * x + (1.0 - x2) * y
    den = 1.0 + 2.0 * xy + x2 * y2
    return num / jnp.clip(den, MIN_NORM, None)


def _pre_body(x_ref, wt_ref, b_ref, o_ref):
    x = x_ref[...]
    mx = jnp.dot(x, wt_ref[...], preferred_element_type=jnp.float32)
    xn = _norm(x)
    mxn = _norm(mx)
    res_c = jnp.tanh(mxn / xn * _artanh(xn)) * mx / mxn
    cond = jnp.all(mx == 0.0, axis=-1, keepdims=True)
    res = _proj(jnp.where(cond, jnp.zeros_like(res_c), res_c))
    hyp_bias = _proj(_expmap0(b_ref[...]))
    res = _proj(_mobius_add(res, hyp_bias))
    o_ref[...] = _logmap0(res)


_pre = pl.pallas_call(
    _pre_body,
    grid=(N // BM,),
    in_specs=[
        pl.BlockSpec((BM, D), lambda i: (i, 0)),
        pl.BlockSpec((D, D), lambda i: (0, 0)),
        pl.BlockSpec((1, D), lambda i: (0, 0)),
    ],
    out_specs=pl.BlockSpec((BM, D), lambda i: (i, 0)),
    out_shape=jax.ShapeDtypeStruct((N, D), jnp.float32),
)


def _post_body(p0_ref, p1_ref, o_ref):
    s = p0_ref[...][0] + p1_ref[...][0]
    h = _proj(_expmap0(s))
    xt = jnp.maximum(_logmap0(h), 0.0)
    o_ref[...] = _proj(_expmap0(xt))


_post = pl.pallas_call(
    _post_body,
    grid=(N // BM,),
    in_specs=[
        pl.BlockSpec((1, BM, D), lambda i: (0, i, 0)),
        pl.BlockSpec((1, BM, D), lambda i: (1, i, 0)),
    ],
    out_specs=pl.BlockSpec((BM, D), lambda i: (i, 0)),
    out_shape=jax.ShapeDtypeStruct((N, D), jnp.float32),
)


def _spmm_body(xt_hbm, src_hbm, dst_hbm, adj_hbm, out_hbm,
               src4, dst4, adj4, rows0, rows1,
               work_v, acc_sh, sg0, sg1):
    c = lax.axis_index("c")
    s = lax.axis_index("s")

    zeros16 = jnp.zeros((16,), jnp.float32)
    for r in range(BE):
        for h in range(D // 16):
            work_v[r, pl.ds(h * 16, 16)] = zeros16
    for z in range(ROWS_PT // BE):
        pltpu.sync_copy(work_v, acc_sh.at[pl.ds(s * ROWS_PT + z * BE, BE)])
    rem = ROWS_PT - (ROWS_PT // BE) * BE
    if rem:
        pltpu.sync_copy(
            work_v.at[pl.ds(0, rem)],
            acc_sh.at[pl.ds(s * ROWS_PT + (ROWS_PT // BE) * BE, rem)])
    plsc.subcore_barrier()

    def scale_and_scatter(rows_v, blk, dst_ref):
        for g in range(BE // 16):
            av = adj4[blk, pl.ds(g * 16, 16)]
            for u in range(16):
                r = g * 16 + u
                a = jnp.full((16,), av[u], jnp.float32)
                for h in range(D // 16):
                    work_v[r, pl.ds(h * 16, 16)] = (
                        rows_v[r, pl.ds(h * 16, 16)] * a)
        pltpu.sync_copy(work_v, acc_sh.at[dst_ref], add=True)

    def quad(q, carry):
        pltpu.sync_copy(src_hbm.at[c, s, q], src4)
        pltpu.sync_copy(dst_hbm.at[c, s, q], dst4)
        pltpu.sync_copy(adj_hbm.at[c, s, q], adj4)
        g0 = pltpu.async_copy(xt_hbm.at[src4.at[0]], rows0, sg0)
        g1 = pltpu.async_copy(xt_hbm.at[src4.at[1]], rows1, sg1)
        g0.wait()
        scale_and_scatter(rows0, 0, dst4.at[0])
        g2 = pltpu.async_copy(xt_hbm.at[src4.at[2]], rows0, sg0)
        g1.wait()
        scale_and_scatter(rows1, 1, dst4.at[1])
        g3 = pltpu.async_copy(xt_hbm.at[src4.at[3]], rows1, sg1)
        g2.wait()
        scale_and_scatter(rows0, 2, dst4.at[2])
        g3.wait()
        scale_and_scatter(rows1, 3, dst4.at[3])
        return carry

    lax.fori_loop(0, QUADS, quad, 0)
    plsc.subcore_barrier()
    pltpu.sync_copy(acc_sh.at[pl.ds(s * ROWS_PT, ROWS_PT)],
                    out_hbm.at[c, pl.ds(s * ROWS_PT, ROWS_PT)])


@functools.lru_cache(maxsize=None)
def _get_spmm():
    return functools.partial(
        pl.kernel,
        out_type=jax.ShapeDtypeStruct((NC, NPAD, D), jnp.float32),
        mesh=plsc.VectorSubcoreMesh(core_axis_name="c", subcore_axis_name="s"),
        scratch_types=[
            pltpu.VMEM((4, BE), jnp.int32),
            pltpu.VMEM((4, BE), jnp.int32),
            pltpu.VMEM((4, BE), jnp.float32),
            pltpu.VMEM((BE, D), jnp.float32),
            pltpu.VMEM((BE, D), jnp.float32),
            pltpu.VMEM((BE, D), jnp.float32),
            pltpu.VMEM_SHARED((NPAD, D), jnp.float32),
            pltpu.SemaphoreType.DMA,
            pltpu.SemaphoreType.DMA,
        ],
    )(_spmm_body)


def kernel(x, edge_index, adj_values, W, b):
    xt = _pre(x, W.T, b.reshape(1, D))
    npad = NC * NS * EPWP - E
    src = jnp.pad(edge_index[0], (0, npad)).reshape(NC, NS, QUADS, 4, BE)
    dst = jnp.pad(edge_index[1], (0, npad)).reshape(NC, NS, QUADS, 4, BE)
    adj = jnp.pad(adj_values, (0, npad)).reshape(NC, NS, QUADS, 4, BE)
    partials = _get_spmm()(xt, src, dst, adj)
    return _post(partials, partials)
